# baseline (speedup 1.0000x reference)
import functools

import jax
import jax.numpy as jnp
from jax import lax
from jax.experimental import pallas as pl
from jax.experimental.pallas import tpu as pltpu
from jax.experimental.pallas import tpu_sc as plsc

N = 10000
D = 256
C = 128
E = 160000
KNN_K = 16

NP = 10112
CH = 128
EP = 163840
NTILES = 16
ROWS_PT = NP // NTILES
NCD = EP // (2 * NTILES * CH)
DW = 16

_mesh = plsc.VectorSubcoreMesh(core_axis_name="c", subcore_axis_name="s")


def _a_body(x_ref, we_ref, be_ref, wa_ref, ba_ref, hl_ref, g_ref):
    x = x_ref[...].astype(jnp.bfloat16)
    hl_ref[...] = jnp.dot(x, we_ref[...].astype(jnp.bfloat16),
                          preferred_element_type=jnp.float32) + be_ref[...]
    g_ref[...] = jnp.dot(x, wa_ref[...].astype(jnp.bfloat16),
                         preferred_element_type=jnp.float32) + ba_ref[...]


def _tc_a(x, W_emb, b_emb, W_ass, b_ass):
    blk = 400
    return pl.pallas_call(
        _a_body,
        grid=(N // blk,),
        in_specs=[
            pl.BlockSpec((blk, D), lambda i: (i, 0)),
            pl.BlockSpec((D, D), lambda i: (0, 0)),
            pl.BlockSpec((1, D), lambda i: (0, 0)),
            pl.BlockSpec((D, C), lambda i: (0, 0)),
            pl.BlockSpec((1, C), lambda i: (0, 0)),
        ],
        out_specs=[
            pl.BlockSpec((blk, D), lambda i: (i, 0)),
            pl.BlockSpec((blk, C), lambda i: (i, 0)),
        ],
        out_shape=[
            jax.ShapeDtypeStruct((N, D), jnp.float32),
            jax.ShapeDtypeStruct((N, C), jnp.float32),
        ],
    )(x, W_emb, b_emb, W_ass, b_ass)


def _deg_body(dstm, zer, out, dstb, onesb, acc, gsem):
    tid = lax.axis_index("s")
    core = lax.axis_index("c")
    crow = core * (NTILES * NCD) + tid * NCD
    pltpu.sync_copy(dstm.at[pl.ds(crow, NCD)], dstb)
    rows0 = tid * ROWS_PT
    pltpu.sync_copy(zer.at[pl.ds(rows0, ROWS_PT)], acc.at[pl.ds(rows0, ROWS_PT)])
    lane = lax.broadcasted_iota(jnp.int32, (16,), 0)
    pat = jnp.where(lane == 0, jnp.float32(1.0), jnp.float32(0.0))
    for r in range(CH):
        onesb[r] = pat
    plsc.subcore_barrier()

    def body(j, carry):
        pltpu.sync_copy(onesb, acc.at[dstb.at[j]], add=True)
        return carry

    lax.fori_loop(0, NCD, body, 0)
    plsc.subcore_barrier()
    pltpu.sync_copy(acc.at[pl.ds(rows0, ROWS_PT)],
                    out.at[core].at[pl.ds(rows0, ROWS_PT)])


def _sc_deg(dst_mat, zeros_d):
    k = functools.partial(
        pl.kernel,
        mesh=_mesh,
        out_type=[jax.ShapeDtypeStruct((2, NP, DW), jnp.float32)],
        scratch_types=[
            pltpu.VMEM((NCD, CH), jnp.int32),
            pltpu.VMEM((CH, DW), jnp.float32),
            pltpu.VMEM_SHARED((NP, DW), jnp.float32),
            pltpu.SemaphoreType.DMA,
        ],
        compiler_params=pltpu.CompilerParams(use_tc_tiling_on_sc=False,
                                             needs_layout_passes=False),
    )(_deg_body)
    return k(dst_mat, zeros_d)


def _b_body(s_ref, adj_ref, t_ref):
    i = pl.program_id(0)
    st_adj = lax.dot_general(s_ref[...].astype(jnp.bfloat16),
                             adj_ref[...].astype(jnp.bfloat16),
                             (((0,), (0,)), ((), ())),
                             preferred_element_type=jnp.float32)

    @pl.when(i == 0)
    def _():
        t_ref[...] = jnp.zeros_like(t_ref)

    t_ref[...] += st_adj


def _tc_b(s, adj_g):
    blk = 200
    return pl.pallas_call(
        _b_body,
        grid=(N // blk,),
        in_specs=[
            pl.BlockSpec((blk, C), lambda i: (i, 0)),
            pl.BlockSpec((blk, N), lambda i: (i, 0)),
        ],
        out_specs=pl.BlockSpec((C, N), lambda i: (0, 0)),
        out_shape=jax.ShapeDtypeStruct((C, N), jnp.float32),
        compiler_params=pltpu.CompilerParams(
            dimension_semantics=("arbitrary",),
        ),
    )(s, adj_g)


def _c_body(t_ref, s_ref, h_ref, w1_ref, b1_ref, w2_ref, b2_ref,
            e_ref, adjg1_ref, me_ref):
    e = lax.dot_general(s_ref[...].astype(jnp.bfloat16),
                        h_ref[...].astype(jnp.bfloat16),
                        (((0,), (0,)), ((), ())),
                        preferred_element_type=jnp.float32)
    e_ref[...] = e
    adjg1_ref[...] = jnp.dot(t_ref[...].astype(jnp.bfloat16),
                             s_ref[...].astype(jnp.bfloat16),
                             preferred_element_type=jnp.float32)
    me = jnp.dot(e.astype(jnp.bfloat16), w1_ref[...].astype(jnp.bfloat16),
                 preferred_element_type=jnp.float32) + b1_ref[...]
    me = jnp.dot(me.astype(jnp.bfloat16), w2_ref[...].astype(jnp.bfloat16),
                 preferred_element_type=jnp.float32) + b2_ref[...]
    me_ref[...] = me


def _tc_c(t, s, h, mlp_W1, mlp_b1, mlp_W2, mlp_b2):
    return pl.pallas_call(
        _c_body,
        out_shape=[
            jax.ShapeDtypeStruct((C, D), jnp.float32),
            jax.ShapeDtypeStruct((C, C), jnp.float32),
            jax.ShapeDtypeStruct((C, D), jnp.float32),
        ],
    )(t, s, h, mlp_W1, mlp_b1, mlp_W2, mlp_b2)


def _d_body(me_ref, sq_ref, adjf1_ref):
    meb = me_ref[...].astype(jnp.bfloat16)
    cross = lax.dot_general(meb, meb, (((1,), (1,)), ((), ())),
                            preferred_element_type=jnp.float32)
    sq = sq_ref[...]
    d2 = sq.T + sq - 2.0 * cross
    val = -d2
    iota_r = lax.broadcasted_iota(jnp.int32, (C, C), 1)
    adj = jnp.zeros((C, C), jnp.float32)
    for _ in range(KNN_K):
        m = jnp.max(val, axis=1, keepdims=True)
        cand = jnp.where(val == m, iota_r, C)
        amin = jnp.min(cand, axis=1, keepdims=True)
        pick = iota_r == amin
        adj = adj + pick.astype(jnp.float32)
        val = jnp.where(pick, -jnp.inf, val)
    adjf1_ref[...] = (adj + adj.T) * 0.5


def _tc_d(me, sq):
    return pl.pallas_call(
        _d_body,
        out_shape=jax.ShapeDtypeStruct((C, C), jnp.float32),
    )(me, sq)


def kernel(x, adj_g, W_emb, b_emb, W_ass, b_ass, att_W, att_b,
           mlp_W1, mlp_b1, mlp_W2, mlp_b2, edge_index):
    src = edge_index[0]
    dst = edge_index[1]

    hl, g = _tc_a(x, W_emb, b_emb.reshape(1, D), W_ass, b_ass.reshape(1, C))

    pad = EP - E
    dst_p = jnp.concatenate([dst, N + (jnp.arange(pad, dtype=jnp.int32) % 16)])
    dst_mat = dst_p.reshape(EP // CH, CH)
    zeros_d = jnp.zeros((NP, DW), jnp.float32)
    (dacc,) = _sc_deg(dst_mat, zeros_d)
    deg = dacc[0, :N, 0] + dacc[1, :N, 0]

    h = jax.ops.segment_sum(hl[src], dst, num_segments=N) / \
        jnp.maximum(deg, 1.0)[:, None]
    z2 = jnp.concatenate([g[src], g[dst]], axis=1)
    e_att = jax.nn.leaky_relu(z2 @ att_W + att_b)
    m = jax.ops.segment_max(e_att, dst, num_segments=N)
    ex = jnp.exp(e_att - m[dst])
    denom = jax.ops.segment_sum(ex, dst, num_segments=N)
    alpha = ex / jnp.maximum(denom[dst], 1e-38)
    s_pre = jax.ops.segment_sum(alpha * g[src], dst, num_segments=N)
    s = jax.nn.softmax(s_pre, axis=-1)

    t = _tc_b(s, adj_g)
    e, adj_g1, me = _tc_c(t, s, h, mlp_W1, mlp_b1.reshape(1, D),
                          mlp_W2, mlp_b2.reshape(1, D))
    sq = jnp.sum(me * me, axis=1)
    adj_f1 = _tc_d(me, sq.reshape(1, C))
    return (h, e, s, adj_g1, adj_f1)
